# baseline (speedup 1.0000x reference)
import functools

import jax
import jax.numpy as jnp
from jax import lax
from jax.experimental import pallas as pl
from jax.experimental.pallas import tpu as pltpu
from jax.experimental.pallas import tpu_sc as plsc

N = 196608
NUM_WORKERS = 32
R = 32
G_BLOCKS = N // (NUM_WORKERS * R)
NBLK = N // R
OUT_W = 1111

SEG_C = [0, 291, 342, 393, 444, 495, 649, 803, 957]
SEG_W = [291, 51, 51, 51, 51, 154, 154, 154, 154]
SEG_P = [c % 8 for c in SEG_C]
SEG_A = [c - p for c, p in zip(SEG_C, SEG_P)]
SEG_WW = [SEG_A[k + 1] - SEG_A[k] for k in range(8)] + [OUT_W - SEG_A[8]]
SEG_BW = []
for k in range(9):
    need = SEG_P[k] + SEG_W[k]
    if k < 8:
        need = max(need, SEG_A[k + 1] - SEG_A[k] + 16)
    if need % 8 and k < 8:
        need += 8 - need % 8
    SEG_BW.append(need)
G8_W = 160
SEG_TW = SEG_BW[:8] + [G8_W]


def _sc_body(*refs):
    tabs = refs[0:9]
    pk_ids, ab_ids, it_ids, mv_ids = refs[9:13]
    out = refs[13]
    idx_s = refs[14]
    abuf = refs[15]
    mbuf = refs[16]
    spans = refs[17:26]
    g8 = refs[26]
    isem = refs[27:29]
    gsem = refs[29:31]
    wsem = refs[31:33]

    wid = lax.axis_index("s") * 2 + lax.axis_index("c")
    base_blk = wid * G_BLOCKS
    lane = lax.iota(jnp.int32, 16)
    lane3 = lane * 3
    lane4 = lane * 4

    def issue_gathers(s, g):
        blk = base_blk + g
        row0 = blk * R
        cps = [
            pltpu.async_copy(pk_ids.at[pl.ds(row0, R)], idx_s.at[s, 0], isem[s]),
            pltpu.async_copy(it_ids.at[pl.ds(row0, R)], idx_s.at[s, 4], isem[s]),
            pltpu.async_copy(ab_ids.at[pl.ds(row0 * 3, 3 * R)], abuf.at[s], isem[s]),
            pltpu.async_copy(mv_ids.at[pl.ds(row0 * 4, 4 * R)], mbuf.at[s], isem[s]),
        ]
        for cp in cps:
            cp.wait()
        for j in range(3):
            for h in range(R // 16):
                idx_s[s, 1 + j, pl.ds(16 * h, 16)] = plsc.load_gather(
                    abuf.at[s], [lane3 + (48 * h + j)])
        for j in range(4):
            for h in range(R // 16):
                idx_s[s, 5 + j, pl.ds(16 * h, 16)] = plsc.load_gather(
                    mbuf.at[s], [lane4 + (64 * h + j)])
        for k in range(8):
            pltpu.async_copy(tabs[k].at[idx_s.at[s, k]], spans[k].at[s], gsem[s])
        pltpu.async_copy(tabs[8].at[idx_s.at[s, 8]], g8.at[s], gsem[s])

    def wait_gathers(s):
        for k in range(8):
            pltpu.make_async_copy(tabs[k].at[pl.ds(0, R)], spans[k].at[s],
                                  gsem[s]).wait()
        pltpu.make_async_copy(tabs[8].at[pl.ds(0, R)], g8.at[s], gsem[s]).wait()

    def fix(s):
        def fr(r, c):
            for col in range(0, 144, 16):
                spans[8][s, r, pl.ds(col, 16)] = g8[s, r, pl.ds(col, 16)]
            spans[8][s, r, pl.ds(143, 16)] = g8[s, r, pl.ds(143, 16)]
            for k in range(1, 9):
                off = SEG_A[k] - SEG_A[k - 1]
                prev = spans[k - 1][s, r, pl.ds(off, 16)]
                cur = spans[k][s, r, pl.ds(0, 16)]
                spans[k][s, r, pl.ds(0, 16)] = jnp.where(lane < SEG_P[k], prev, cur)
            return c
        lax.fori_loop(0, R, fr, 0)

    def issue_writes(s, g):
        row0 = (base_blk + g) * R
        for k in range(9):
            pltpu.async_copy(
                spans[k].at[s, pl.ds(0, R), pl.ds(0, SEG_WW[k])],
                out.at[pl.ds(row0, R), pl.ds(SEG_A[k], SEG_WW[k])], wsem[s])

    def wait_writes(s):
        for k in range(9):
            pltpu.make_async_copy(
                spans[k].at[s, pl.ds(0, R), pl.ds(0, SEG_WW[k])],
                out.at[pl.ds(0, R), pl.ds(SEG_A[k], SEG_WW[k])], wsem[s]).wait()

    issue_gathers(0, 0)

    def pair(h, c):
        for s in (0, 1):
            g = 2 * h + s
            o = 1 - s

            @pl.when(g + 1 < G_BLOCKS)
            def _():
                if s == 0:
                    @pl.when(h >= 1)
                    def _():
                        wait_writes(o)
                else:
                    wait_writes(o)
                issue_gathers(o, g + 1)

            wait_gathers(s)
            fix(s)
            issue_writes(s, g)
        return c

    lax.fori_loop(0, G_BLOCKS // 2, pair, 0)
    wait_writes(0)
    wait_writes(1)


@jax.jit
def _sc_encode(tabs, pk_ids, ab_ids, it_ids, mv_ids):
    mesh = plsc.VectorSubcoreMesh(core_axis_name="c", subcore_axis_name="s")
    kern = functools.partial(
        pl.kernel,
        mesh=mesh,
        out_type=jax.ShapeDtypeStruct((N, OUT_W), jnp.float32),
        scratch_types=[pltpu.VMEM((2, 9, R), jnp.int32),
                       pltpu.VMEM((2, 3 * R), jnp.int32),
                       pltpu.VMEM((2, 4 * R), jnp.int32)]
        + [pltpu.VMEM((2, R, SEG_BW[k]), jnp.float32) for k in range(9)]
        + [pltpu.VMEM((2, R, G8_W), jnp.float32)]
        + [pltpu.SemaphoreType.DMA] * 6,
        compiler_params=pltpu.CompilerParams(use_tc_tiling_on_sc=False,
                                             needs_layout_passes=False),
    )(_sc_body)
    return kern(*tabs, pk_ids, ab_ids, it_ids, mv_ids)


def _pad_tab(tab, k):
    left = SEG_P[k]
    right = SEG_TW[k] - SEG_P[k] - SEG_W[k]
    return jnp.pad(tab, ((0, 0), (left, right)))


def kernel(pokemon_ids, ability_ids, item_ids, move_ids,
           pokemon_static, pokemon_learn, ability_static, ability_learn,
           item_static, item_learn, move_static, move_learn):
    pk_tab = jnp.concatenate([pokemon_static, pokemon_learn], axis=1)
    ab_tab = jnp.concatenate([ability_static, ability_learn], axis=1)
    it_tab = jnp.concatenate([item_static, item_learn], axis=1)
    mv_tab = jnp.concatenate([move_static, move_learn], axis=1)
    src = [pk_tab, ab_tab, ab_tab, ab_tab, it_tab, mv_tab, mv_tab, mv_tab, mv_tab]
    tabs = [_pad_tab(src[k], k) for k in range(9)]
    return _sc_encode(tabs, pokemon_ids, ability_ids.reshape(-1),
                      item_ids.reshape(-1), move_ids.reshape(-1))


# speedup vs baseline: 1.0418x; 1.0418x over previous
import functools

import jax
import jax.numpy as jnp
from jax import lax
from jax.experimental import pallas as pl
from jax.experimental.pallas import tpu as pltpu
from jax.experimental.pallas import tpu_sc as plsc

N = 196608
NUM_WORKERS = 32
R = 32
G_BLOCKS = N // (NUM_WORKERS * R)
OUT_W = 1111

SEG_C = [0, 291, 342, 393, 444, 495, 649, 803, 957]
SEG_W = [291, 51, 51, 51, 51, 154, 154, 154, 154]
SEG_P = [c % 8 for c in SEG_C]
SEG_A = [c - p for c, p in zip(SEG_C, SEG_P)]
SEG_WW = [SEG_A[k + 1] - SEG_A[k] for k in range(8)] + [152]
PK_BW = 304
AI_BW = 72
MV_BW = 200
NV_AB, NV_IT, NV_MV = 311, 249, 686


def _sc_body(*refs):
    t_pk, t_ai, t_mv = refs[0:3]
    pk_ids, ab_ids, it_ids, mv_ids = refs[3:7]
    out = refs[7]
    idx_s = refs[8]
    abuf = refs[9]
    mbuf = refs[10]
    pbuf = refs[11]
    ibuf = refs[12]
    s_pk = refs[13]
    s_ai = refs[14]
    s_mv = refs[15]
    t159 = refs[16]
    isem = refs[17:19]
    gsem = refs[19:21]
    wsem = refs[21:23]

    wid = lax.axis_index("s") * 2 + lax.axis_index("c")
    base_blk = wid * G_BLOCKS
    lane = lax.iota(jnp.int32, 16)
    lane3 = lane * 3
    lane4 = lane * 4
    H = R // 16

    def seg(s, k, r):
        if k == 0:
            return s_pk.at[s, r]
        if k <= 4:
            return s_ai.at[s, (k - 1) * R + r]
        return s_mv.at[s, (k - 5) * R + r]

    def issue_idx(s, g):
        row0 = (base_blk + g) * R
        pltpu.async_copy(pk_ids.at[pl.ds(row0, R)], pbuf.at[s], isem[s])
        pltpu.async_copy(it_ids.at[pl.ds(row0, R)], ibuf.at[s], isem[s])
        pltpu.async_copy(ab_ids.at[pl.ds(row0 * 3, 3 * R)], abuf.at[s], isem[s])
        pltpu.async_copy(mv_ids.at[pl.ds(row0 * 4, 4 * R)], mbuf.at[s], isem[s])

    def wait_idx(s):
        pltpu.make_async_copy(pk_ids.at[pl.ds(0, R)], pbuf.at[s], isem[s]).wait()
        pltpu.make_async_copy(it_ids.at[pl.ds(0, R)], ibuf.at[s], isem[s]).wait()
        pltpu.make_async_copy(ab_ids.at[pl.ds(0, 3 * R)], abuf.at[s], isem[s]).wait()
        pltpu.make_async_copy(mv_ids.at[pl.ds(0, 4 * R)], mbuf.at[s], isem[s]).wait()

    def issue_gathers(s):
        for h in range(H):
            idx_s[s, pl.ds(16 * h, 16)] = pbuf[s, pl.ds(16 * h, 16)]
            idx_s[s, pl.ds(4 * R + 16 * h, 16)] = (
                ibuf[s, pl.ds(16 * h, 16)] + 3 * NV_AB)
        for j in range(3):
            for h in range(H):
                idx_s[s, pl.ds(R + j * R + 16 * h, 16)] = plsc.load_gather(
                    abuf.at[s], [lane3 + (48 * h + j)]) + NV_AB * j
        for j in range(4):
            for h in range(H):
                idx_s[s, pl.ds(5 * R + j * R + 16 * h, 16)] = plsc.load_gather(
                    mbuf.at[s], [lane4 + (64 * h + j)]) + NV_MV * j
        pltpu.async_copy(t_pk.at[idx_s.at[s, pl.ds(0, R)]], s_pk.at[s], gsem[s])
        pltpu.async_copy(t_ai.at[idx_s.at[s, pl.ds(R, 4 * R)]], s_ai.at[s], gsem[s])
        pltpu.async_copy(t_mv.at[idx_s.at[s, pl.ds(5 * R, 4 * R)]], s_mv.at[s], gsem[s])

    def wait_gathers(s):
        pltpu.make_async_copy(t_pk.at[pl.ds(0, R)], s_pk.at[s], gsem[s]).wait()
        pltpu.make_async_copy(t_ai.at[pl.ds(0, 4 * R)], s_ai.at[s], gsem[s]).wait()
        pltpu.make_async_copy(t_mv.at[pl.ds(0, 4 * R)], s_mv.at[s], gsem[s]).wait()

    def fix(s):
        def fr(r, c):
            t159[s, r, pl.ds(143, 16)] = s_mv[s, 3 * R + r, pl.ds(143, 16)]
            for k in range(1, 9):
                off = SEG_A[k] - SEG_A[k - 1]
                prev = seg(s, k - 1, r)[pl.ds(off, 16)]
                cur = seg(s, k, r)[pl.ds(0, 16)]
                seg(s, k, r)[pl.ds(0, 16)] = jnp.where(lane < SEG_P[k], prev, cur)
            return c
        lax.fori_loop(0, R, fr, 0)

    def _write_list(s, row0):
        dsts = [(s_pk.at[s, pl.ds(0, R), pl.ds(0, SEG_WW[0])],
                 out.at[pl.ds(row0, R), pl.ds(SEG_A[0], SEG_WW[0])])]
        for k in range(1, 5):
            dsts.append((s_ai.at[s, pl.ds((k - 1) * R, R), pl.ds(0, SEG_WW[k])],
                         out.at[pl.ds(row0, R), pl.ds(SEG_A[k], SEG_WW[k])]))
        for k in range(5, 9):
            dsts.append((s_mv.at[s, pl.ds((k - 5) * R, R), pl.ds(0, SEG_WW[k])],
                         out.at[pl.ds(row0, R), pl.ds(SEG_A[k], SEG_WW[k])]))
        dsts.append((t159.at[s, pl.ds(0, R), pl.ds(152, 7)],
                     out.at[pl.ds(row0, R), pl.ds(1104, 7)]))
        return dsts

    def issue_writes(s, g):
        row0 = (base_blk + g) * R
        for src, dst in _write_list(s, row0):
            pltpu.async_copy(src, dst, wsem[s])

    def wait_writes(s):
        for src, dst in _write_list(s, 0):
            pltpu.make_async_copy(src, dst, wsem[s]).wait()

    issue_idx(0, 0)
    issue_idx(1, 1)
    wait_idx(0)
    issue_gathers(0)

    def pair(h, c):
        for s in (0, 1):
            g = 2 * h + s
            o = 1 - s

            @pl.when(g + 2 < G_BLOCKS)
            def _():
                issue_idx(s, g + 2)

            @pl.when(g + 1 < G_BLOCKS)
            def _():
                if s == 0:
                    @pl.when(h >= 1)
                    def _():
                        wait_writes(o)
                else:
                    wait_writes(o)
                wait_idx(o)
                issue_gathers(o)

            wait_gathers(s)
            fix(s)
            issue_writes(s, g)
        return c

    lax.fori_loop(0, G_BLOCKS // 2, pair, 0)
    wait_writes(0)
    wait_writes(1)


@jax.jit
def _sc_encode(t_pk, t_ai, t_mv, pk_ids, ab_ids, it_ids, mv_ids):
    mesh = plsc.VectorSubcoreMesh(core_axis_name="c", subcore_axis_name="s")
    kern = functools.partial(
        pl.kernel,
        mesh=mesh,
        out_type=jax.ShapeDtypeStruct((N, OUT_W), jnp.float32),
        scratch_types=[pltpu.VMEM((2, 9 * R), jnp.int32),
                       pltpu.VMEM((2, 3 * R), jnp.int32),
                       pltpu.VMEM((2, 4 * R), jnp.int32),
                       pltpu.VMEM((2, R), jnp.int32),
                       pltpu.VMEM((2, R), jnp.int32),
                       pltpu.VMEM((2, R, PK_BW), jnp.float32),
                       pltpu.VMEM((2, 4 * R, AI_BW), jnp.float32),
                       pltpu.VMEM((2, 4 * R, MV_BW), jnp.float32),
                       pltpu.VMEM((2, R, 159), jnp.float32)]
        + [pltpu.SemaphoreType.DMA] * 6,
        compiler_params=pltpu.CompilerParams(use_tc_tiling_on_sc=False,
                                             needs_layout_passes=False),
    )(_sc_body)
    return kern(t_pk, t_ai, t_mv, pk_ids, ab_ids, it_ids, mv_ids)


def _pad(tab, left, width):
    return jnp.pad(tab, ((0, 0), (left, width - left - tab.shape[1])))


def kernel(pokemon_ids, ability_ids, item_ids, move_ids,
           pokemon_static, pokemon_learn, ability_static, ability_learn,
           item_static, item_learn, move_static, move_learn):
    pk_tab = jnp.concatenate([pokemon_static, pokemon_learn], axis=1)
    ab_tab = jnp.concatenate([ability_static, ability_learn], axis=1)
    it_tab = jnp.concatenate([item_static, item_learn], axis=1)
    mv_tab = jnp.concatenate([move_static, move_learn], axis=1)
    t_pk = _pad(pk_tab, SEG_P[0], PK_BW)
    t_ai = jnp.concatenate(
        [_pad(ab_tab, SEG_P[1], AI_BW), _pad(ab_tab, SEG_P[2], AI_BW),
         _pad(ab_tab, SEG_P[3], AI_BW), _pad(it_tab, SEG_P[4], AI_BW)], axis=0)
    t_mv = jnp.concatenate(
        [_pad(mv_tab, SEG_P[5], MV_BW), _pad(mv_tab, SEG_P[6], MV_BW),
         _pad(mv_tab, SEG_P[7], MV_BW), _pad(mv_tab, SEG_P[8], MV_BW)], axis=0)
    return _sc_encode(t_pk, t_ai, t_mv, pokemon_ids, ability_ids.reshape(-1),
                      item_ids.reshape(-1), move_ids.reshape(-1))


# speedup vs baseline: 1.1071x; 1.0626x over previous
import functools

import jax
import jax.numpy as jnp
from jax import lax
from jax.experimental import pallas as pl
from jax.experimental.pallas import tpu as pltpu
from jax.experimental.pallas import tpu_sc as plsc

N = 196608
NUM_WORKERS = 32
R = 32
G_BLOCKS = N // (NUM_WORKERS * R)
NBLK = N // R
OUT_W = 1111

SEG_C = [0, 291, 342, 393, 444, 495, 649, 803, 957]
SEG_W = [291, 51, 51, 51, 51, 154, 154, 154, 154]
SEG_P = [c % 8 for c in SEG_C]
SEG_A = [c - p for c, p in zip(SEG_C, SEG_P)]
SEG_WW = [SEG_A[k + 1] - SEG_A[k] for k in range(8)] + [152]
PK_BW = 304
AI_BW = 72
MV_BW = 200
NV_AB, NV_MV = 311, 686


def _sc_body(*refs):
    t_pk, t_ai, t_mv, idxs, out = refs[0:5]
    idx_s = refs[5]
    s_pk = refs[6]
    s_ai = refs[7]
    s_mv = refs[8]
    t159 = refs[9]
    gsem = refs[10:12]
    wsem = refs[12:14]

    wid = lax.axis_index("s") * 2 + lax.axis_index("c")
    base_blk = wid * G_BLOCKS
    lane = lax.iota(jnp.int32, 16)

    def seg(s, k, r):
        if k == 0:
            return s_pk.at[s, r]
        if k <= 4:
            return s_ai.at[s, (k - 1) * R + r]
        return s_mv.at[s, (k - 5) * R + r]

    def issue_gathers(s, g):
        blk = base_blk + g
        pltpu.sync_copy(idxs.at[blk], idx_s.at[s])
        pltpu.async_copy(t_pk.at[idx_s.at[s, pl.ds(0, R)]], s_pk.at[s], gsem[s])
        pltpu.async_copy(t_ai.at[idx_s.at[s, pl.ds(R, 4 * R)]], s_ai.at[s], gsem[s])
        pltpu.async_copy(t_mv.at[idx_s.at[s, pl.ds(5 * R, 4 * R)]], s_mv.at[s], gsem[s])

    def wait_gathers(s):
        pltpu.make_async_copy(t_pk.at[pl.ds(0, R)], s_pk.at[s], gsem[s]).wait()
        pltpu.make_async_copy(t_ai.at[pl.ds(0, 4 * R)], s_ai.at[s], gsem[s]).wait()
        pltpu.make_async_copy(t_mv.at[pl.ds(0, 4 * R)], s_mv.at[s], gsem[s]).wait()

    def fix(s):
        def fr(r, c):
            t159[s, r, pl.ds(143, 16)] = s_mv[s, 3 * R + r, pl.ds(143, 16)]
            for k in range(1, 9):
                off = SEG_A[k] - SEG_A[k - 1]
                prev = seg(s, k - 1, r)[pl.ds(off, 16)]
                cur = seg(s, k, r)[pl.ds(0, 16)]
                seg(s, k, r)[pl.ds(0, 16)] = jnp.where(lane < SEG_P[k], prev, cur)
            return c
        lax.fori_loop(0, R, fr, 0)

    def _write_list(s, row0):
        dsts = [(s_pk.at[s, pl.ds(0, R), pl.ds(0, SEG_WW[0])],
                 out.at[pl.ds(row0, R), pl.ds(SEG_A[0], SEG_WW[0])])]
        for k in range(1, 5):
            dsts.append((s_ai.at[s, pl.ds((k - 1) * R, R), pl.ds(0, SEG_WW[k])],
                         out.at[pl.ds(row0, R), pl.ds(SEG_A[k], SEG_WW[k])]))
        for k in range(5, 9):
            dsts.append((s_mv.at[s, pl.ds((k - 5) * R, R), pl.ds(0, SEG_WW[k])],
                         out.at[pl.ds(row0, R), pl.ds(SEG_A[k], SEG_WW[k])]))
        dsts.append((t159.at[s, pl.ds(0, R), pl.ds(152, 7)],
                     out.at[pl.ds(row0, R), pl.ds(1104, 7)]))
        return dsts

    def issue_writes(s, g):
        row0 = (base_blk + g) * R
        for src, dst in _write_list(s, row0):
            pltpu.async_copy(src, dst, wsem[s])

    def wait_writes(s):
        for src, dst in _write_list(s, 0):
            pltpu.make_async_copy(src, dst, wsem[s]).wait()

    issue_gathers(0, 0)

    def pair(h, c):
        for s in (0, 1):
            g = 2 * h + s
            o = 1 - s

            @pl.when(g + 1 < G_BLOCKS)
            def _():
                if s == 0:
                    @pl.when(h >= 1)
                    def _():
                        wait_writes(o)
                else:
                    wait_writes(o)
                issue_gathers(o, g + 1)

            wait_gathers(s)
            fix(s)
            issue_writes(s, g)
        return c

    lax.fori_loop(0, G_BLOCKS // 2, pair, 0)
    wait_writes(0)
    wait_writes(1)


@jax.jit
def _sc_encode(t_pk, t_ai, t_mv, idxs):
    mesh = plsc.VectorSubcoreMesh(core_axis_name="c", subcore_axis_name="s")
    kern = functools.partial(
        pl.kernel,
        mesh=mesh,
        out_type=jax.ShapeDtypeStruct((N, OUT_W), jnp.float32),
        scratch_types=[pltpu.VMEM((2, 9 * R), jnp.int32),
                       pltpu.VMEM((2, R, PK_BW), jnp.float32),
                       pltpu.VMEM((2, 4 * R, AI_BW), jnp.float32),
                       pltpu.VMEM((2, 4 * R, MV_BW), jnp.float32),
                       pltpu.VMEM((2, R, 159), jnp.float32)]
        + [pltpu.SemaphoreType.DMA] * 4,
        compiler_params=pltpu.CompilerParams(use_tc_tiling_on_sc=False,
                                             needs_layout_passes=False),
    )(_sc_body)
    return kern(t_pk, t_ai, t_mv, idxs)


def _pad(tab, left, width):
    return jnp.pad(tab, ((0, 0), (left, width - left - tab.shape[1])))


def kernel(pokemon_ids, ability_ids, item_ids, move_ids,
           pokemon_static, pokemon_learn, ability_static, ability_learn,
           item_static, item_learn, move_static, move_learn):
    pk_tab = jnp.concatenate([pokemon_static, pokemon_learn], axis=1)
    ab_tab = jnp.concatenate([ability_static, ability_learn], axis=1)
    it_tab = jnp.concatenate([item_static, item_learn], axis=1)
    mv_tab = jnp.concatenate([move_static, move_learn], axis=1)
    t_pk = _pad(pk_tab, SEG_P[0], PK_BW)
    t_ai = jnp.concatenate(
        [_pad(ab_tab, SEG_P[1], AI_BW), _pad(ab_tab, SEG_P[2], AI_BW),
         _pad(ab_tab, SEG_P[3], AI_BW), _pad(it_tab, SEG_P[4], AI_BW)], axis=0)
    t_mv = jnp.concatenate(
        [_pad(mv_tab, SEG_P[5], MV_BW), _pad(mv_tab, SEG_P[6], MV_BW),
         _pad(mv_tab, SEG_P[7], MV_BW), _pad(mv_tab, SEG_P[8], MV_BW)], axis=0)
    idx_all = jnp.concatenate(
        [pokemon_ids[:, None],
         ability_ids + jnp.arange(3, dtype=jnp.int32) * NV_AB,
         item_ids + 3 * NV_AB,
         move_ids + jnp.arange(4, dtype=jnp.int32) * NV_MV], axis=1)
    idxs = idx_all.T.reshape(9, NBLK, R).transpose(1, 0, 2).reshape(NBLK, 9 * R)
    return _sc_encode(t_pk, t_ai, t_mv, idxs)
